# baseline (speedup 1.0000x reference)
import functools

import jax
import jax.numpy as jnp
from jax import lax
from jax.experimental import pallas as pl
from jax.experimental.pallas import tpu as pltpu
from jax.experimental.pallas import tpu_sc as plsc

VOCAB = 1000000
EMB = 64
EMBP = 128
NCLS = 20
L = 200
B = 4096

_R = 512
_NBLK = (VOCAB + 2 * _R - 1) // (2 * _R)
_NPACK = _NBLK * _R
_NROWS = 2 * _NPACK

_info = plsc.get_sparse_core_info()
_NC, _NS = _info.num_cores, _info.num_subcores
_NW = _NC * _NS
_BPW = B // _NW
_VPR = EMB // 16


def _tc_pack(table):
    tab_t = table.T

    def body(a_ref, b_ref, o_ref):
        o_ref[...] = jnp.concatenate(
            [a_ref[...].T, b_ref[...].T], axis=1)

    return pl.pallas_call(
        body,
        grid=(_NBLK,),
        in_specs=[
            pl.BlockSpec((EMB, _R), lambda i: (0, 2 * i)),
            pl.BlockSpec((EMB, _R), lambda i: (0, 2 * i + 1)),
        ],
        out_specs=pl.BlockSpec((_R, EMBP), lambda i: (i, 0)),
        out_shape=jax.ShapeDtypeStruct((_NPACK, EMBP), jnp.float32),
    )(tab_t, tab_t)


def _sc_embed_sum(docs32, table_lin):
    mesh = plsc.VectorSubcoreMesh(core_axis_name="c", subcore_axis_name="s")

    @functools.partial(
        pl.kernel,
        mesh=mesh,
        out_type=jax.ShapeDtypeStruct((B, EMB), jnp.float32),
        scratch_types=[
            pltpu.VMEM((L, _BPW), jnp.int32),
            pltpu.VMEM((_BPW, EMB), jnp.float32),
            pltpu.VMEM((_BPW, EMB), jnp.float32),
            pltpu.VMEM((_BPW, EMB), jnp.float32),
            pltpu.SemaphoreType.DMA,
            pltpu.SemaphoreType.DMA,
        ],
        compiler_params=pltpu.CompilerParams(use_tc_tiling_on_sc=False),
    )
    def k(docs_hbm, table_hbm, out_hbm, idx_v, buf0, buf1, acc, sem0, sem1):
        wid = lax.axis_index("s") * _NC + lax.axis_index("c")
        base = wid * _BPW
        pltpu.sync_copy(docs_hbm.at[:, pl.ds(base, _BPW)], idx_v)

        def xform_row(r, _):
            for c in range(_BPW // 16):
                s = pl.ds(c * 16, 16)
                t = idx_v[r, s]
                row = (
                    (t & jnp.int32(~1023))
                    | ((t & jnp.int32(511)) << 1)
                    | ((t >> 9) & jnp.int32(1))
                )
                idx_v[r, s] = row
            return 0

        lax.fori_loop(0, L, xform_row, 0, unroll=4)

        def accum(buf, first):
            def row_body(r, _):
                for c in range(_VPR):
                    s = pl.ds(c * 16, 16)
                    x = buf[r, s]
                    if first:
                        acc[r, s] = x
                    else:
                        plsc.addupdate(acc.at[r, s], x)
                return 0
            lax.fori_loop(0, _BPW, row_body, 0, unroll=4)

        pltpu.async_copy(table_hbm.at[idx_v.at[0]], buf0, sem0)

        def pair_body(lp, _):
            l0 = 2 * lp
            pltpu.make_async_copy(table_hbm.at[idx_v.at[l0]], buf0, sem0).wait()
            pltpu.async_copy(table_hbm.at[idx_v.at[l0 + 1]], buf1, sem1)
            accum(buf0, first=False)
            pltpu.make_async_copy(table_hbm.at[idx_v.at[l0 + 1]], buf1, sem1).wait()

            @pl.when(lp < (L // 2) - 1)
            def _():
                pltpu.async_copy(table_hbm.at[idx_v.at[l0 + 2]], buf0, sem0)

            accum(buf1, first=False)
            return 0

        pltpu.make_async_copy(table_hbm.at[idx_v.at[0]], buf0, sem0).wait()
        pltpu.async_copy(table_hbm.at[idx_v.at[1]], buf1, sem1)
        accum(buf0, first=True)
        pltpu.make_async_copy(table_hbm.at[idx_v.at[1]], buf1, sem1).wait()
        pltpu.async_copy(table_hbm.at[idx_v.at[2]], buf0, sem0)
        accum(buf1, first=False)

        lax.fori_loop(1, L // 2, pair_body, 0)

        pltpu.sync_copy(acc, out_hbm.at[pl.ds(base, _BPW)])

    return k(docs32, table_lin)


def _tc_classifier(sums, W, b):
    def body(x_ref, w_ref, b_ref, o_ref):
        x = x_ref[...] * (1.0 / L)
        o_ref[...] = (
            lax.dot_general(x, w_ref[...], (((1,), (1,)), ((), ())),
                            preferred_element_type=jnp.float32)
            + b_ref[...]
        )

    return pl.pallas_call(
        body,
        out_shape=jax.ShapeDtypeStruct((B, NCLS), jnp.float32),
    )(sums, W, b.reshape(1, NCLS))


def kernel(docs, table, W, b):
    docs32 = docs.astype(jnp.int32)
    packed = _tc_pack(table)
    table_lin = packed.reshape(_NROWS, EMB)
    sums = _sc_embed_sum(docs32, table_lin)
    return _tc_classifier(sums, W, b)


# speedup vs baseline: 1.6583x; 1.6583x over previous
import functools

import jax
import jax.numpy as jnp
from jax import lax
from jax.experimental import pallas as pl
from jax.experimental.pallas import tpu as pltpu
from jax.experimental.pallas import tpu_sc as plsc

VOCAB = 1000000
EMB = 64
EMBP = 128
NCLS = 20
L = 200
B = 4096

_R = 2048
_NBLK = (VOCAB + 2 * _R - 1) // (2 * _R)
_NPACK = _NBLK * _R
_NROWS = 2 * _NPACK

_info = plsc.get_sparse_core_info()
_NC, _NS = _info.num_cores, _info.num_subcores
_NW = _NC * _NS
_BPW = B // _NW
_VPR = EMB // 16


def _tc_pack(table):
    tab_t = table.T

    def body(a_ref, b_ref, o_ref):
        ii = lax.broadcasted_iota(jnp.int32, (EMB, EMB), 0)
        jj = lax.broadcasted_iota(jnp.int32, (EMB, EMB), 1)
        ident = (ii == jj).astype(jnp.float32)
        at = lax.dot_general(a_ref[...], ident, (((0,), (0,)), ((), ())),
                             preferred_element_type=jnp.float32)
        bt = lax.dot_general(b_ref[...], ident, (((0,), (0,)), ((), ())),
                             preferred_element_type=jnp.float32)
        o_ref[...] = jnp.concatenate([at, bt], axis=1)

    last_blk = (VOCAB - 1) // _R

    return pl.pallas_call(
        body,
        grid=(_NBLK,),
        in_specs=[
            pl.BlockSpec((EMB, _R), lambda i: (0, jnp.minimum(2 * i, last_blk))),
            pl.BlockSpec((EMB, _R),
                         lambda i: (0, jnp.minimum(2 * i + 1, last_blk))),
        ],
        out_specs=pl.BlockSpec((_R, EMBP), lambda i: (i, 0)),
        out_shape=jax.ShapeDtypeStruct((_NPACK, EMBP), jnp.float32),
    )(tab_t, tab_t)


def _sc_embed_sum(docs32, table_lin):
    mesh = plsc.VectorSubcoreMesh(core_axis_name="c", subcore_axis_name="s")

    @functools.partial(
        pl.kernel,
        mesh=mesh,
        out_type=jax.ShapeDtypeStruct((B, EMB), jnp.float32),
        scratch_types=[
            pltpu.VMEM((L, _BPW), jnp.int32),
            pltpu.VMEM((_BPW, EMB), jnp.float32),
            pltpu.VMEM((_BPW, EMB), jnp.float32),
            pltpu.VMEM((_BPW, EMB), jnp.float32),
            pltpu.SemaphoreType.DMA,
            pltpu.SemaphoreType.DMA,
        ],
        compiler_params=pltpu.CompilerParams(use_tc_tiling_on_sc=False),
    )
    def k(docs_hbm, table_hbm, out_hbm, idx_v, buf0, buf1, acc, sem0, sem1):
        wid = lax.axis_index("s") * _NC + lax.axis_index("c")
        base = wid * _BPW
        pltpu.sync_copy(docs_hbm.at[:, pl.ds(base, _BPW)], idx_v)

        sh = _R.bit_length() - 1

        def xform_row(r, _):
            for c in range(_BPW // 16):
                s = pl.ds(c * 16, 16)
                t = idx_v[r, s]
                row = (
                    (t & jnp.int32(~(2 * _R - 1)))
                    | ((t & jnp.int32(_R - 1)) << 1)
                    | ((t >> sh) & jnp.int32(1))
                )
                idx_v[r, s] = row
            return 0

        lax.fori_loop(0, L, xform_row, 0, unroll=4)

        def accum(buf, first):
            def row_body(r, _):
                for c in range(_VPR):
                    s = pl.ds(c * 16, 16)
                    x = buf[r, s]
                    if first:
                        acc[r, s] = x
                    else:
                        plsc.addupdate(acc.at[r, s], x)
                return 0
            lax.fori_loop(0, _BPW, row_body, 0, unroll=4)

        pltpu.async_copy(table_hbm.at[idx_v.at[0]], buf0, sem0)

        def pair_body(lp, _):
            l0 = 2 * lp
            pltpu.make_async_copy(table_hbm.at[idx_v.at[l0]], buf0, sem0).wait()
            pltpu.async_copy(table_hbm.at[idx_v.at[l0 + 1]], buf1, sem1)
            accum(buf0, first=False)
            pltpu.make_async_copy(table_hbm.at[idx_v.at[l0 + 1]], buf1, sem1).wait()

            @pl.when(lp < (L // 2) - 1)
            def _():
                pltpu.async_copy(table_hbm.at[idx_v.at[l0 + 2]], buf0, sem0)

            accum(buf1, first=False)
            return 0

        pltpu.make_async_copy(table_hbm.at[idx_v.at[0]], buf0, sem0).wait()
        pltpu.async_copy(table_hbm.at[idx_v.at[1]], buf1, sem1)
        accum(buf0, first=True)
        pltpu.make_async_copy(table_hbm.at[idx_v.at[1]], buf1, sem1).wait()
        pltpu.async_copy(table_hbm.at[idx_v.at[2]], buf0, sem0)
        accum(buf1, first=False)

        lax.fori_loop(1, L // 2, pair_body, 0)

        pltpu.sync_copy(acc, out_hbm.at[pl.ds(base, _BPW)])

    return k(docs32, table_lin)


def _tc_classifier(sums, W, b):
    def body(x_ref, w_ref, b_ref, o_ref):
        x = x_ref[...] * (1.0 / L)
        o_ref[...] = (
            lax.dot_general(x, w_ref[...], (((1,), (1,)), ((), ())),
                            preferred_element_type=jnp.float32)
            + b_ref[...]
        )

    return pl.pallas_call(
        body,
        out_shape=jax.ShapeDtypeStruct((B, NCLS), jnp.float32),
    )(sums, W, b.reshape(1, NCLS))


def kernel(docs, table, W, b):
    docs32 = docs.astype(jnp.int32)
    packed = _tc_pack(table)
    table_lin = packed.reshape(_NROWS, EMB)
    sums = _sc_embed_sum(docs32, table_lin)
    return _tc_classifier(sums, W, b)


# speedup vs baseline: 1.7481x; 1.0542x over previous
import functools

import jax
import jax.numpy as jnp
from jax import lax
from jax.experimental import pallas as pl
from jax.experimental.pallas import tpu as pltpu
from jax.experimental.pallas import tpu_sc as plsc

VOCAB = 1000000
EMB = 64
NCLS = 20
L = 200
B = 4096

_R = 2048
_SH = _R.bit_length() - 1
_NBLK = (VOCAB + 4 * _R - 1) // (4 * _R)
_NPACK = _NBLK * _R
_NROWS = 4 * _NPACK

_info = plsc.get_sparse_core_info()
_NC, _NS = _info.num_cores, _info.num_subcores
_NW = _NC * _NS
_BPW = B // _NW


def _tc_pack(table):
    tab_t = table.T
    last_blk = (VOCAB - 1) // _R

    def body(a_ref, b_ref, c_ref, d_ref, o_ref):
        ii = lax.broadcasted_iota(jnp.int32, (EMB, EMB), 0)
        jj = lax.broadcasted_iota(jnp.int32, (EMB, EMB), 1)
        ident = (ii == jj).astype(jnp.float32)

        def pack_one(ref):
            at = lax.dot_general(ref[...], ident, (((0,), (0,)), ((), ())),
                                 preferred_element_type=jnp.float32)
            bits = lax.bitcast_convert_type(at, jnp.uint32)
            lo = (bits[:, :32] + jnp.uint32(0x8000)) >> 16
            hi = (bits[:, 32:] + jnp.uint32(0x8000)) & jnp.uint32(0xFFFF0000)
            return lax.bitcast_convert_type(lo | hi, jnp.int32)

        o_ref[...] = jnp.concatenate(
            [pack_one(a_ref), pack_one(b_ref), pack_one(c_ref),
             pack_one(d_ref)], axis=1)

    def mk_spec(q):
        return pl.BlockSpec(
            (EMB, _R), lambda i: (0, jnp.minimum(4 * i + q, last_blk)))

    return pl.pallas_call(
        body,
        grid=(_NBLK,),
        in_specs=[mk_spec(0), mk_spec(1), mk_spec(2), mk_spec(3)],
        out_specs=pl.BlockSpec((_R, 128), lambda i: (i, 0)),
        out_shape=jax.ShapeDtypeStruct((_NPACK, 128), jnp.int32),
    )(tab_t, tab_t, tab_t, tab_t)


def _sc_embed_sum(docs32, table_lin):
    mesh = plsc.VectorSubcoreMesh(core_axis_name="c", subcore_axis_name="s")
    nbuf = 4

    @functools.partial(
        pl.kernel,
        mesh=mesh,
        out_type=jax.ShapeDtypeStruct((B, EMB), jnp.float32),
        scratch_types=[
            pltpu.VMEM((L, _BPW), jnp.int32),
            pltpu.VMEM((nbuf, _BPW, 32), jnp.int32),
            pltpu.VMEM((_BPW, EMB), jnp.float32),
            pltpu.SemaphoreType.DMA,
            pltpu.SemaphoreType.DMA,
            pltpu.SemaphoreType.DMA,
            pltpu.SemaphoreType.DMA,
        ],
        compiler_params=pltpu.CompilerParams(use_tc_tiling_on_sc=False),
    )
    def k(docs_hbm, table_hbm, out_hbm, idx_v, bufs, acc, *sems):
        wid = lax.axis_index("s") * _NC + lax.axis_index("c")
        base = wid * _BPW
        pltpu.sync_copy(docs_hbm.at[:, pl.ds(base, _BPW)], idx_v)

        def xform_row(r, _):
            for c in range(_BPW // 16):
                s = pl.ds(c * 16, 16)
                t = idx_v[r, s]
                row = (
                    (t & jnp.int32(~(4 * _R - 1)))
                    | ((t & jnp.int32(_R - 1)) << 2)
                    | ((t >> _SH) & jnp.int32(3))
                )
                idx_v[r, s] = row
            return 0

        lax.fori_loop(0, L, xform_row, 0, unroll=4)

        himask = jnp.int32(-65536)

        def unpack2(x):
            lo = lax.bitcast_convert_type(x << 16, jnp.float32)
            hi = lax.bitcast_convert_type(x & himask, jnp.float32)
            return lo, hi

        def accum_pair(ba, bb, first):
            def row_body(r, _):
                for c in range(2):
                    s = pl.ds(c * 16, 16)
                    alo, ahi = unpack2(ba[r, s])
                    blo, bhi = unpack2(bb[r, s])
                    lo = alo + blo
                    hi = ahi + bhi
                    slo = pl.ds(c * 16, 16)
                    shi = pl.ds(32 + c * 16, 16)
                    if first:
                        acc[r, slo] = lo
                        acc[r, shi] = hi
                    else:
                        plsc.addupdate(acc.at[r, slo], lo)
                        plsc.addupdate(acc.at[r, shi], hi)
                return 0
            lax.fori_loop(0, _BPW, row_body, 0, unroll=4)

        def issue(tok, bslot):
            pltpu.async_copy(
                table_hbm.at[idx_v.at[tok]], bufs.at[bslot], sems[bslot])

        def wait(tok, bslot):
            pltpu.make_async_copy(
                table_hbm.at[idx_v.at[tok]], bufs.at[bslot],
                sems[bslot]).wait()

        for q in range(nbuf):
            issue(q, q)

        def quad(j, first):
            t0 = 4 * j
            wait(t0, 0)
            wait(t0 + 1, 1)
            accum_pair(bufs.at[0], bufs.at[1], first)

            @pl.when(j < (L // 4) - 1)
            def _():
                issue(t0 + 4, 0)
                issue(t0 + 5, 1)

            wait(t0 + 2, 2)
            wait(t0 + 3, 3)
            accum_pair(bufs.at[2], bufs.at[3], False)

            @pl.when(j < (L // 4) - 1)
            def _():
                issue(t0 + 6, 2)
                issue(t0 + 7, 3)

        quad(0, True)
        lax.fori_loop(1, L // 4, lambda j, _: (quad(j, False), 0)[1], 0)

        pltpu.sync_copy(acc, out_hbm.at[pl.ds(base, _BPW)])

    return k(docs32, table_lin)


def _tc_classifier(sums, W, b):
    def body(x_ref, w_ref, b_ref, o_ref):
        x = x_ref[...] * (1.0 / L)
        o_ref[...] = (
            lax.dot_general(x, w_ref[...], (((1,), (1,)), ((), ())),
                            preferred_element_type=jnp.float32)
            + b_ref[...]
        )

    return pl.pallas_call(
        body,
        out_shape=jax.ShapeDtypeStruct((B, NCLS), jnp.float32),
    )(sums, W, b.reshape(1, NCLS))


def kernel(docs, table, W, b):
    docs32 = docs.astype(jnp.int32)
    packed = _tc_pack(table)
    table_lin = packed.reshape(_NROWS, 32)
    sums = _sc_embed_sum(docs32, table_lin)
    return _tc_classifier(sums, W, b)
